# baseline (speedup 1.0000x reference)
import jax
import jax.numpy as jnp
from jax.experimental import pallas as pl
from jax.experimental.pallas import tpu as pltpu

_KEEP_RATIO = 0.1
_B = 64
_H = 512
_W = 512
_N = _H * _W
_K = max(1, int(_N * _KEEP_RATIO))
_S = 4


def _softplus_neg(a):
    u = jnp.exp(-a)
    q = jnp.float32(-0.023689253892508606)
    q = q * u + jnp.float32(0.10028720563794943)
    q = q * u + jnp.float32(-0.20866966052158784)
    q = q * u + jnp.float32(0.3244118094075504)
    q = q * u + jnp.float32(-0.4991878509930077)
    q = q * u + jnp.float32(0.999981872180961)
    return u * q


def _bce_with_logits(logits, targets):
    return (jnp.maximum(logits, 0.0) - logits * targets
            + _softplus_neg(jnp.abs(logits)))


def _topk_kernel(logits_ref, targets_ref, out_ref, loss_ref, tprev_ref):
    step = pl.program_id(0)
    loss_ref[...] = jnp.maximum(
        _bce_with_logits(logits_ref[...], targets_ref[...]),
        0.0).astype(jnp.bfloat16)

    loss = loss_ref[...]
    axes = (1, 2, 3)
    bf_one = jnp.bfloat16(1.0)
    bf_zero = jnp.bfloat16(0.0)

    def _edge(code):
        f32 = jax.lax.bitcast_convert_type(code << 16, jnp.float32)
        return f32.astype(jnp.bfloat16)

    def _count_ge(thr):
        mask = jnp.where(loss >= thr, bf_one, bf_zero)
        part = jnp.sum(mask.reshape(_S, 1, 2, _H // 2, _W), axis=3,
                       dtype=jnp.bfloat16)
        return jnp.sum(part.astype(jnp.float32), axis=(1, 2, 3),
                       keepdims=False).reshape(_S, 1, 1, 1)

    lo_f = jnp.min(loss, axis=axes, keepdims=True).astype(jnp.float32)
    hi_f = jnp.max(loss, axis=axes, keepdims=True).astype(jnp.float32)
    lo0 = (jax.lax.bitcast_convert_type(lo_f, jnp.int32) >> 16) - 1
    hi0 = jax.lax.bitcast_convert_type(hi_f, jnp.int32) >> 16

    def cond(carry):
        lo, hi = carry
        return jnp.max(hi - lo) > 1

    tp = tprev_ref[...]
    a = jnp.maximum(jnp.min(tp) - 32, lo0)
    b = jnp.minimum(jnp.max(tp) + 32, hi0)
    ca = _count_ge(_edge(a + 1))
    cb = _count_ge(_edge(b + 1))
    good = jnp.logical_and(
        step > 0,
        jnp.logical_and(ca >= jnp.float32(_K), cb < jnp.float32(_K)))
    lo_init = jnp.where(good, a, lo0)
    hi_init = jnp.where(good, b, hi0)

    def body(carry):
        lo, hi = carry
        mid = lo + jax.lax.shift_right_logical(hi - lo, 1)
        cnt = _count_ge(_edge(mid + 1))
        keep_lo = cnt >= jnp.float32(_K)
        lo = jnp.where(keep_lo, mid, lo)
        hi = jnp.where(keep_lo, hi, mid)
        return lo, hi

    _, tcode = jax.lax.while_loop(cond, body, (lo_init, hi_init))
    tprev_ref[...] = tcode

    ub = _edge(tcode + 1)
    ge = loss >= ub
    cnt_ge = _count_ge(ub).reshape(_S, 1, 1, 1)
    sum_ge = jnp.sum(jnp.where(ge, loss, bf_zero).astype(jnp.float32),
                     axis=axes, keepdims=True)
    tval = jax.lax.bitcast_convert_type(tcode << 16, jnp.float32)
    part = jnp.sum(sum_ge + tval * (jnp.float32(_K) - cnt_ge))
    out_ref[...] = jnp.reshape(part, (1, 1, 1))


def kernel(logits, targets):
    out = pl.pallas_call(
        _topk_kernel,
        grid=(_B // _S,),
        in_specs=[
            pl.BlockSpec((_S, 1, _H, _W), lambda b: (b, 0, 0, 0)),
            pl.BlockSpec((_S, 1, _H, _W), lambda b: (b, 0, 0, 0)),
        ],
        out_specs=pl.BlockSpec((1, 1, 1), lambda b: (b, 0, 0)),
        out_shape=jax.ShapeDtypeStruct((_B // _S, 1, 1), jnp.float32),
        scratch_shapes=[pltpu.VMEM((_S, 1, _H, _W), jnp.bfloat16),
                        pltpu.VMEM((_S, 1, 1, 1), jnp.int32)],
        compiler_params=pltpu.CompilerParams(
            dimension_semantics=("arbitrary",)),
    )(logits, targets)
    return jnp.sum(out) / jnp.float32(_B * _K)


# speedup vs baseline: 1.2368x; 1.2368x over previous
import jax
import jax.numpy as jnp
from jax.experimental import pallas as pl
from jax.experimental.pallas import tpu as pltpu

_KEEP_RATIO = 0.1
_B = 64
_H = 512
_W = 512
_N = _H * _W
_K = max(1, int(_N * _KEEP_RATIO))
_S = 8


def _bce_with_logits(logits, targets):
    return (jnp.maximum(logits, 0.0) - logits * targets
            + jnp.log1p(jnp.exp(-jnp.abs(logits))))


def _topk_kernel(logits_ref, targets_ref, out_ref, loss_ref, tprev_ref):
    step = pl.program_id(0)
    loss_ref[...] = jnp.maximum(
        _bce_with_logits(logits_ref[...], targets_ref[...]),
        0.0).astype(jnp.bfloat16)

    loss = loss_ref[...]
    axes = (1, 2, 3)
    bf_one = jnp.bfloat16(1.0)
    bf_zero = jnp.bfloat16(0.0)

    def _edge(code):
        f32 = jax.lax.bitcast_convert_type(code << 16, jnp.float32)
        return f32.astype(jnp.bfloat16)

    def _count_ge(thr):
        mask = jnp.where(loss >= thr, bf_one, bf_zero)
        part = jnp.sum(mask.reshape(_S, 1, 2, _H // 2, _W), axis=3,
                       dtype=jnp.bfloat16)
        return jnp.sum(part.astype(jnp.float32), axis=(1, 2, 3),
                       keepdims=False).reshape(_S, 1, 1, 1)

    lo_f = jnp.min(loss, axis=axes, keepdims=True).astype(jnp.float32)
    hi_f = jnp.max(loss, axis=axes, keepdims=True).astype(jnp.float32)
    lo0 = (jax.lax.bitcast_convert_type(lo_f, jnp.int32) >> 16) - 1
    hi0 = jax.lax.bitcast_convert_type(hi_f, jnp.int32) >> 16

    def cond(carry):
        lo, hi = carry
        return jnp.max(hi - lo) > 1

    tp = tprev_ref[...]
    a = jnp.maximum(jnp.min(tp) - 32, lo0)
    b = jnp.minimum(jnp.max(tp) + 32, hi0)
    ca = _count_ge(_edge(a + 1))
    cb = _count_ge(_edge(b + 1))
    good = jnp.logical_and(
        step > 0,
        jnp.logical_and(ca >= jnp.float32(_K), cb < jnp.float32(_K)))
    lo_init = jnp.where(good, a, lo0)
    hi_init = jnp.where(good, b, hi0)

    def body(carry):
        lo, hi = carry
        mid = lo + jax.lax.shift_right_logical(hi - lo, 1)
        cnt = _count_ge(_edge(mid + 1))
        keep_lo = cnt >= jnp.float32(_K)
        lo = jnp.where(keep_lo, mid, lo)
        hi = jnp.where(keep_lo, hi, mid)
        return lo, hi

    _, tcode = jax.lax.while_loop(cond, body, (lo_init, hi_init))
    tprev_ref[...] = tcode

    ub = _edge(tcode + 1)
    ge = loss >= ub
    cnt_ge = _count_ge(ub).reshape(_S, 1, 1, 1)
    sum_ge = jnp.sum(jnp.where(ge, loss, bf_zero).astype(jnp.float32),
                     axis=axes, keepdims=True)
    tval = jax.lax.bitcast_convert_type(tcode << 16, jnp.float32)
    part = jnp.sum(sum_ge + tval * (jnp.float32(_K) - cnt_ge))
    out_ref[...] = jnp.reshape(part, (1, 1, 1))


def kernel(logits, targets):
    out = pl.pallas_call(
        _topk_kernel,
        grid=(_B // _S,),
        in_specs=[
            pl.BlockSpec((_S, 1, _H, _W), lambda b: (b, 0, 0, 0)),
            pl.BlockSpec((_S, 1, _H, _W), lambda b: (b, 0, 0, 0)),
        ],
        out_specs=pl.BlockSpec((1, 1, 1), lambda b: (b, 0, 0)),
        out_shape=jax.ShapeDtypeStruct((_B // _S, 1, 1), jnp.float32),
        scratch_shapes=[pltpu.VMEM((_S, 1, _H, _W), jnp.bfloat16),
                        pltpu.VMEM((_S, 1, 1, 1), jnp.int32)],
        compiler_params=pltpu.CompilerParams(
            dimension_semantics=("arbitrary",)),
    )(logits, targets)
    return jnp.sum(out) / jnp.float32(_B * _K)


# speedup vs baseline: 1.2875x; 1.0409x over previous
import jax
import jax.numpy as jnp
from jax.experimental import pallas as pl
from jax.experimental.pallas import tpu as pltpu

_KEEP_RATIO = 0.1
_B = 64
_H = 512
_W = 512
_N = _H * _W
_K = max(1, int(_N * _KEEP_RATIO))
_S = 8


def _bce_with_logits(logits, targets):
    return (jnp.maximum(logits, 0.0) - logits * targets
            + jnp.log1p(jnp.exp(-jnp.abs(logits))))


def _topk_kernel(logits_ref, targets_ref, out_ref, loss_ref, tprev_ref):
    step = pl.program_id(0)
    loss_ref[...] = jnp.maximum(
        _bce_with_logits(logits_ref[...], targets_ref[...]),
        0.0).astype(jnp.bfloat16)

    loss = loss_ref[...]
    axes = (1, 2, 3)
    bf_one = jnp.bfloat16(1.0)
    bf_zero = jnp.bfloat16(0.0)

    def _edge(code):
        f32 = jax.lax.bitcast_convert_type(code << 16, jnp.float32)
        return f32.astype(jnp.bfloat16)

    def _count_ge(thr):
        mask = jnp.where(loss >= thr, bf_one, bf_zero)
        part = jnp.sum(mask.reshape(_S, 1, 2, _H // 2, _W), axis=3,
                       dtype=jnp.bfloat16)
        return jnp.sum(part.astype(jnp.float32), axis=(1, 2, 3),
                       keepdims=False).reshape(_S, 1, 1, 1)

    lo_f = jnp.min(loss, axis=axes, keepdims=True).astype(jnp.float32)
    hi_f = jnp.max(loss, axis=axes, keepdims=True).astype(jnp.float32)
    lo0 = (jax.lax.bitcast_convert_type(lo_f, jnp.int32) >> 16) - 1
    hi0 = jax.lax.bitcast_convert_type(hi_f, jnp.int32) >> 16

    def cond(carry):
        lo, hi = carry
        return jnp.max(hi - lo) > 1

    tp = tprev_ref[...]
    a = jnp.maximum(jnp.min(tp) - 16, lo0)
    b = jnp.minimum(jnp.max(tp) + 16, hi0)
    ca = _count_ge(_edge(a + 1))
    cb = _count_ge(_edge(b + 1))
    good = jnp.logical_and(
        step > 0,
        jnp.logical_and(ca >= jnp.float32(_K), cb < jnp.float32(_K)))
    lo_init = jnp.where(good, a, lo0)
    hi_init = jnp.where(good, b, hi0)

    def body(carry):
        lo, hi = carry
        mid = lo + jax.lax.shift_right_logical(hi - lo, 1)
        cnt = _count_ge(_edge(mid + 1))
        keep_lo = cnt >= jnp.float32(_K)
        lo = jnp.where(keep_lo, mid, lo)
        hi = jnp.where(keep_lo, hi, mid)
        return lo, hi

    _, tcode = jax.lax.while_loop(cond, body, (lo_init, hi_init))
    tprev_ref[...] = tcode

    ub = _edge(tcode + 1)
    ge = loss >= ub
    cnt_ge = _count_ge(ub).reshape(_S, 1, 1, 1)
    sum_ge = jnp.sum(jnp.where(ge, loss, bf_zero).astype(jnp.float32),
                     axis=axes, keepdims=True)
    tval = jax.lax.bitcast_convert_type(tcode << 16, jnp.float32)
    part = jnp.sum(sum_ge + tval * (jnp.float32(_K) - cnt_ge))
    out_ref[...] = jnp.reshape(part, (1, 1, 1))


def kernel(logits, targets):
    out = pl.pallas_call(
        _topk_kernel,
        grid=(_B // _S,),
        in_specs=[
            pl.BlockSpec((_S, 1, _H, _W), lambda b: (b, 0, 0, 0)),
            pl.BlockSpec((_S, 1, _H, _W), lambda b: (b, 0, 0, 0)),
        ],
        out_specs=pl.BlockSpec((1, 1, 1), lambda b: (b, 0, 0)),
        out_shape=jax.ShapeDtypeStruct((_B // _S, 1, 1), jnp.float32),
        scratch_shapes=[pltpu.VMEM((_S, 1, _H, _W), jnp.bfloat16),
                        pltpu.VMEM((_S, 1, 1, 1), jnp.int32)],
        compiler_params=pltpu.CompilerParams(
            dimension_semantics=("arbitrary",)),
    )(logits, targets)
    return jnp.sum(out) / jnp.float32(_B * _K)


# speedup vs baseline: 1.3424x; 1.0427x over previous
import jax
import jax.numpy as jnp
from jax.experimental import pallas as pl
from jax.experimental.pallas import tpu as pltpu

_KEEP_RATIO = 0.1
_B = 64
_H = 512
_W = 512
_N = _H * _W
_K = max(1, int(_N * _KEEP_RATIO))
_S = 8


def _bce_with_logits(logits, targets):
    return (jnp.maximum(logits, 0.0) - logits * targets
            + jnp.log1p(jnp.exp(-jnp.abs(logits))))


def _topk_kernel(logits_ref, targets_ref, out_ref, loss_ref, tprev_ref):
    step = pl.program_id(0)
    loss_ref[...] = jnp.maximum(
        _bce_with_logits(logits_ref[...], targets_ref[...]),
        0.0).astype(jnp.bfloat16)

    loss = loss_ref[...]
    axes = (1, 2, 3)
    bf_one = jnp.bfloat16(1.0)
    bf_zero = jnp.bfloat16(0.0)

    def _edge(code):
        f32 = jax.lax.bitcast_convert_type(code << 16, jnp.float32)
        return f32.astype(jnp.bfloat16)

    def _count_ge(thr):
        mask = jnp.where(loss >= thr, bf_one, bf_zero)
        part = jnp.sum(mask.reshape(_S, 1, 2, _H // 2, _W), axis=3,
                       dtype=jnp.bfloat16)
        return jnp.sum(part.astype(jnp.float32), axis=(1, 2, 3),
                       keepdims=False).reshape(_S, 1, 1, 1)

    lo_f = jnp.min(loss, axis=axes, keepdims=True).astype(jnp.float32)
    hi_f = jnp.max(loss, axis=axes, keepdims=True).astype(jnp.float32)
    lo0 = (jax.lax.bitcast_convert_type(lo_f, jnp.int32) >> 16) - 1
    hi0 = jax.lax.bitcast_convert_type(hi_f, jnp.int32) >> 16

    def cond(carry):
        lo, hi = carry
        return jnp.max(hi - lo) > 1

    tp = tprev_ref[...]
    a = jnp.maximum(jnp.min(tp) - 8, lo0)
    b = jnp.minimum(jnp.max(tp) + 8, hi0)
    ca = _count_ge(_edge(a + 1))
    cb = _count_ge(_edge(b + 1))
    good = jnp.logical_and(
        step > 0,
        jnp.logical_and(ca >= jnp.float32(_K), cb < jnp.float32(_K)))
    lo_init = jnp.where(good, a, lo0)
    hi_init = jnp.where(good, b, hi0)

    def body(carry):
        lo, hi = carry
        mid = lo + jax.lax.shift_right_logical(hi - lo, 1)
        cnt = _count_ge(_edge(mid + 1))
        keep_lo = cnt >= jnp.float32(_K)
        lo = jnp.where(keep_lo, mid, lo)
        hi = jnp.where(keep_lo, hi, mid)
        return lo, hi

    _, tcode = jax.lax.while_loop(cond, body, (lo_init, hi_init))
    tprev_ref[...] = tcode

    ub = _edge(tcode + 1)
    ge = loss >= ub
    cnt_ge = _count_ge(ub).reshape(_S, 1, 1, 1)
    sum_ge = jnp.sum(jnp.where(ge, loss, bf_zero).astype(jnp.float32),
                     axis=axes, keepdims=True)
    tval = jax.lax.bitcast_convert_type(tcode << 16, jnp.float32)
    part = jnp.sum(sum_ge + tval * (jnp.float32(_K) - cnt_ge))
    out_ref[...] = jnp.reshape(part, (1, 1, 1))


def kernel(logits, targets):
    out = pl.pallas_call(
        _topk_kernel,
        grid=(_B // _S,),
        in_specs=[
            pl.BlockSpec((_S, 1, _H, _W), lambda b: (b, 0, 0, 0)),
            pl.BlockSpec((_S, 1, _H, _W), lambda b: (b, 0, 0, 0)),
        ],
        out_specs=pl.BlockSpec((1, 1, 1), lambda b: (b, 0, 0)),
        out_shape=jax.ShapeDtypeStruct((_B // _S, 1, 1), jnp.float32),
        scratch_shapes=[pltpu.VMEM((_S, 1, _H, _W), jnp.bfloat16),
                        pltpu.VMEM((_S, 1, 1, 1), jnp.int32)],
        compiler_params=pltpu.CompilerParams(
            dimension_semantics=("arbitrary",)),
    )(logits, targets)
    return jnp.sum(out) / jnp.float32(_B * _K)
